# baseline (speedup 1.0000x reference)
import functools

import jax
import jax.numpy as jnp
from jax import lax
from jax.experimental import pallas as pl
from jax.experimental.pallas import tpu as pltpu
from jax.experimental.pallas import tpu_sc as plsc

_NENTITY = 1000000
_EMBED_DIM = 16
_N_VEC = 4
_BATCH = 16384
_ROW = _EMBED_DIM * _N_VEC

_NW = 32
_RPW = _BATCH // _NW
_CHUNK = 128
_NCHUNK = _RPW // _CHUNK
_GROUPS = _RPW * _N_VEC // 16


def _rsqrt16(x):
    i = plsc.bitcast(x, jnp.int32)
    i = 0x5F3759DF - (i >> 1)
    y = plsc.bitcast(i, jnp.float32)
    xh = x * 0.5
    for _ in range(3):
        y = y * (1.5 - xh * y * y)
    return y


@functools.partial(
    pl.kernel,
    mesh=plsc.VectorSubcoreMesh(core_axis_name="c", subcore_axis_name="s"),
    out_type=jax.ShapeDtypeStruct((_BATCH, _ROW), jnp.float32),
    scratch_types=[
        pltpu.VMEM((_NCHUNK, _CHUNK), jnp.int32),
        pltpu.VMEM((_RPW, _ROW), jnp.float32),
        pltpu.VMEM((_RPW, _ROW), jnp.float32),
        pltpu.SemaphoreType.DMA,
    ],
    compiler_params=pltpu.CompilerParams(
        needs_layout_passes=False, use_tc_tiling_on_sc=False
    ),
)
def _encode(idx_hbm, table_hbm, out_hbm, idx_v, rows_v, out_v, sem):
    wid = lax.axis_index("s") * 2 + lax.axis_index("c")
    base = wid * _RPW

    pltpu.sync_copy(idx_hbm.at[pl.ds(wid * _NCHUNK, _NCHUNK)], idx_v)
    copies = [
        pltpu.async_copy(
            table_hbm.at[idx_v.at[j]],
            rows_v.at[pl.ds(j * _CHUNK, _CHUNK)],
            sem,
        )
        for j in range(_NCHUNK)
    ]
    for cp in copies:
        cp.wait()

    lane = lax.iota(jnp.int32, 16)
    rowoff = lane >> 2
    coloff = (lane & 3) * 16

    def group_body(grp, carry):
        row_idx = rowoff + grp * 4
        vs = []
        acc = None
        for p in range(16):
            v = plsc.load_gather(rows_v, [row_idx, coloff + p])
            vs.append(v)
            sq = v * v
            acc = sq if acc is None else acc + sq
        scale = _rsqrt16(acc)
        for p in range(16):
            plsc.store_scatter(out_v, [row_idx, coloff + p], vs[p] * scale)
        return carry

    lax.fori_loop(0, _GROUPS, group_body, 0)

    pltpu.sync_copy(out_v, out_hbm.at[pl.ds(base, _RPW)])


def kernel(indices, table):
    idx = indices.astype(jnp.int32).reshape(_BATCH // _CHUNK, _CHUNK)
    out = _encode(idx, table)
    return out.reshape(_BATCH, _N_VEC, _EMBED_DIM)


# speedup vs baseline: 1.0089x; 1.0089x over previous
import functools

import jax
import jax.numpy as jnp
from jax import lax
from jax.experimental import pallas as pl
from jax.experimental.pallas import tpu as pltpu
from jax.experimental.pallas import tpu_sc as plsc

_NENTITY = 1000000
_EMBED_DIM = 16
_N_VEC = 4
_BATCH = 16384
_ROW = _EMBED_DIM * _N_VEC
_PAIR = 2 * _ROW

_NW = 32
_RPW = _BATCH // _NW
_CHUNK = 128
_NCHUNK = _RPW // _CHUNK
_CGROUPS = _CHUNK * _N_VEC // 16


def _rsqrt16(x):
    i = plsc.bitcast(x, jnp.int32)
    i = 0x5F3759DF - (i >> 1)
    y = plsc.bitcast(i, jnp.float32)
    xh = x * 0.5
    for _ in range(3):
        y = y * (1.5 - xh * y * y)
    return y


@functools.partial(
    pl.kernel,
    mesh=plsc.VectorSubcoreMesh(core_axis_name="c", subcore_axis_name="s"),
    out_type=jax.ShapeDtypeStruct((_BATCH, _ROW), jnp.float32),
    scratch_types=[
        pltpu.VMEM((_NCHUNK, _CHUNK), jnp.int32),
        pltpu.VMEM((_NCHUNK, _CHUNK), jnp.int32),
        pltpu.VMEM((2 * _CHUNK, _PAIR), jnp.float32),
        pltpu.VMEM((_RPW, _ROW), jnp.float32),
        pltpu.SemaphoreType.DMA,
    ],
    compiler_params=pltpu.CompilerParams(needs_layout_passes=False),
)
def _encode(idx_hbm, table_hbm, out_hbm, idx_v, gidx_v, rows_v, out_v, sem):
    wid = lax.axis_index("s") * 2 + lax.axis_index("c")
    base = wid * _RPW

    pltpu.sync_copy(idx_hbm.at[pl.ds(wid * _NCHUNK, _NCHUNK)], idx_v)
    for j in range(_NCHUNK):
        for t in range(_CHUNK // 16):
            gidx_v[j, pl.ds(t * 16, 16)] = idx_v[j, pl.ds(t * 16, 16)] >> 1

    def gather_chunk(j):
        return pltpu.async_copy(
            table_hbm.at[gidx_v.at[j]],
            rows_v.at[pl.ds((j % 2) * _CHUNK, _CHUNK)],
            sem,
        )

    lane = lax.iota(jnp.int32, 16)
    rowoff = lane >> 2
    coloff = (lane & 3) * 16

    def process_chunk(j, copy):
        copy.wait()
        buf_base = (j % 2) * _CHUNK
        out_base = j * _CHUNK

        def group_body(grp, carry):
            local_row = rowoff + grp * 4
            orig = plsc.load_gather(idx_v.at[j], [local_row])
            src_col = coloff + (orig & 1) * _ROW
            row_idx = local_row + buf_base
            vs = []
            acc = None
            for p in range(16):
                v = plsc.load_gather(rows_v, [row_idx, src_col + p])
                vs.append(v)
                sq = v * v
                acc = sq if acc is None else acc + sq
            scale = _rsqrt16(acc)
            out_row = local_row + out_base
            for p in range(16):
                plsc.store_scatter(out_v, [out_row, coloff + p], vs[p] * scale)
            return carry

        lax.fori_loop(0, _CGROUPS, group_body, 0)

    copies = [gather_chunk(0), gather_chunk(1)]
    for j in range(_NCHUNK):
        process_chunk(j, copies[j])
        if j + 2 < _NCHUNK:
            copies.append(gather_chunk(j + 2))

    pltpu.sync_copy(out_v, out_hbm.at[pl.ds(base, _RPW)])


def kernel(indices, table):
    idx = indices.astype(jnp.int32).reshape(_BATCH // _CHUNK, _CHUNK)
    pairs = table.reshape(_NENTITY // 2, _PAIR)
    out = _encode(idx, pairs)
    return out.reshape(_BATCH, _N_VEC, _EMBED_DIM)


# speedup vs baseline: 1.7044x; 1.6894x over previous
import functools

import jax
import jax.numpy as jnp
from jax import lax
from jax.experimental import pallas as pl
from jax.experimental.pallas import tpu as pltpu
from jax.experimental.pallas import tpu_sc as plsc

_NENTITY = 1000000
_EMBED_DIM = 16
_N_VEC = 4
_BATCH = 16384
_ROW = _EMBED_DIM * _N_VEC

_NW = 32
_RPW = _BATCH // _NW
_GROUPS = _RPW * _N_VEC // 16


def _rsqrt16(x):
    i = plsc.bitcast(x, jnp.int32)
    i = 0x5F3759DF - (i >> 1)
    y = plsc.bitcast(i, jnp.float32)
    xh = x * 0.5
    for _ in range(3):
        y = y * (1.5 - xh * y * y)
    return y


@functools.partial(
    pl.kernel,
    mesh=plsc.VectorSubcoreMesh(core_axis_name="c", subcore_axis_name="s"),
    out_type=jax.ShapeDtypeStruct((_BATCH, _ROW), jnp.float32),
    scratch_types=[
        pltpu.VMEM((_RPW,), jnp.int32),
        pltpu.VMEM((_RPW, _ROW), jnp.float32),
        pltpu.SemaphoreType.DMA,
    ],
    compiler_params=pltpu.CompilerParams(needs_layout_passes=False),
)
def _encode(idx_hbm, table_hbm, out_hbm, idx_v, rows_v, sem):
    wid = lax.axis_index("s") * 2 + lax.axis_index("c")
    base = wid * _RPW

    pltpu.sync_copy(idx_hbm.at[pl.ds(base, _RPW)], idx_v)

    def fire_block(t, carry):
        iv = idx_v[pl.ds(t * 16, 16)]
        r0 = t * 16
        for j in range(16):
            s = iv[j]
            pltpu.async_copy(
                table_hbm.at[pl.ds(s, 1)], rows_v.at[pl.ds(r0 + j, 1)], sem
            )
        return carry

    lax.fori_loop(0, _RPW // 16, fire_block, 0)

    pltpu.make_async_copy(
        table_hbm.at[pl.ds(0, _RPW)], rows_v, sem
    ).wait()

    lane = lax.iota(jnp.int32, 16)
    rowoff = lane >> 2
    coloff = (lane & 3) * 16

    def group_body(grp, carry):
        row_idx = rowoff + grp * 4
        vs = []
        acc = None
        for p in range(16):
            v = plsc.load_gather(rows_v, [row_idx, coloff + p])
            vs.append(v)
            sq = v * v
            acc = sq if acc is None else acc + sq
        scale = _rsqrt16(acc)
        for p in range(16):
            plsc.store_scatter(rows_v, [row_idx, coloff + p], vs[p] * scale)
        return carry

    lax.fori_loop(0, _GROUPS, group_body, 0)

    pltpu.sync_copy(rows_v, out_hbm.at[pl.ds(base, _RPW)])


def kernel(indices, table):
    idx = indices.astype(jnp.int32)
    out = _encode(idx, table)
    return out.reshape(_BATCH, _N_VEC, _EMBED_DIM)
